# baseline (speedup 1.0000x reference)
import jax
import jax.numpy as jnp
from jax.experimental import pallas as pl
from jax.experimental.pallas import tpu as pltpu

N1 = 2048
DIM = 64
K = 20
ALPHA = 3.0
NN = 2 * N1
TILE = 128
NT = N1 // TILE


def _dot_t(a, b, precision=jax.lax.Precision.HIGHEST):
    return jax.lax.dot_general(
        a, b, (((1,), (1,)), ((), ())),
        precision=precision,
        preferred_element_type=jnp.float32)


def _nv_kernel(emb_ref, lw_ref, lb_ref, nv1_ref, nv2_ref):
    for r in range(2):
        for j in range(2):
            i1 = 2 * r + j
            i2 = 2 * j + r
            nv1_ref[r, j] = jnp.tanh(
                ALPHA * (_dot_t(emb_ref[i1], lw_ref[i1]) + lb_ref[i1]))
            nv2_ref[r, j] = jnp.tanh(
                ALPHA * (_dot_t(emb_ref[i2], lw_ref[i1]) + lb_ref[i1]))


def _fused(nv1_ref, nv2_ref, ww_ref, wb_ref, pre_ref, out_ref):
    t = pl.program_id(1)

    accw = _dot_t(pre_ref[0], ww_ref[0]) + wb_ref[0]

    row0 = nv1_ref[0, 0, pl.ds(t * TILE, TILE), :]
    row1 = nv1_ref[0, 1, pl.ds(t * TILE, TILE), :]
    log0 = _dot_t(row0, nv2_ref[0, 0]) + accw
    log1 = _dot_t(row1, nv2_ref[0, 1]) + accw
    logits = jnp.concatenate([log0, log1], axis=1)

    adj = jnp.maximum(jnp.tanh(ALPHA * logits), 0.0)

    bits = jax.lax.bitcast_convert_type(adj, jnp.int32)
    ONE = 0x3F800000
    cnt1 = jnp.sum((bits == ONE).astype(jnp.int32), axis=1, keepdims=True)
    lo0 = jnp.where(cnt1 >= K, ONE - 1, -1)
    hi0 = jnp.full((TILE, 1), ONE, jnp.int32)
    n0 = jnp.zeros((TILE, 1), jnp.int32)

    def _cond(state):
        lo, hi, n_gt = state
        return jnp.any(hi - lo > 1)

    def _body(state):
        lo, hi, n_gt = state
        mid = (lo + hi) >> 1
        cnt = jnp.sum((bits > mid).astype(jnp.int32), axis=1, keepdims=True)
        ge = cnt >= K
        return (jnp.where(ge, mid, lo), jnp.where(ge, hi, mid),
                jnp.where(ge, n_gt, cnt))

    _, thr, n_gt = jax.lax.while_loop(_cond, _body, (lo0, hi0, n0))
    m_tie = K - n_gt
    tie = bits == thr
    iota = jax.lax.broadcasted_iota(jnp.int32, (TILE, NN), 1)
    ilo = jnp.full((TILE, 1), -1, jnp.int32)
    ihi = jnp.full((TILE, 1), NN - 1, jnp.int32)
    for _ in range(12):
        mid = (ilo + ihi) >> 1
        c = jnp.sum((tie & (iota <= mid)).astype(jnp.int32), axis=1,
                    keepdims=True)
        ok = c >= m_tie
        ihi = jnp.where(ok, mid, ihi)
        ilo = jnp.where(ok, ilo, mid)
    mask = (bits > thr) | (tie & (iota <= ihi))
    out_ref[...] = jnp.where(mask, adj, 0.0)


def kernel(emb0, emb1, emb2, emb3, lw0, lw1, lw2, lw3, lb0, lb1, lb2, lb3,
           ww0, ww1, wb0, wb1, pre_adj0, pre_adj1, idx):
    emb = jnp.stack([emb0, emb1, emb2, emb3])
    lw = jnp.stack([lw0, lw1, lw2, lw3])
    lb = jnp.stack([lb0, lb1, lb2, lb3])[:, None, :]
    ww = jnp.stack([ww0, ww1])
    wb = jnp.stack([wb0, wb1])[:, None, :]
    pre = jnp.stack([pre_adj0, pre_adj1])

    nv_shape = jax.ShapeDtypeStruct((2, 2, N1, DIM), jnp.float32)
    nv1, nv2 = pl.pallas_call(
        _nv_kernel,
        out_shape=(nv_shape, nv_shape),
    )(emb, lw, lb)

    return pl.pallas_call(
        _fused,
        grid=(2, NT),
        in_specs=[
            pl.BlockSpec((1, 2, N1, DIM), lambda r, t: (r, 0, 0, 0)),
            pl.BlockSpec((1, 2, N1, DIM), lambda r, t: (r, 0, 0, 0)),
            pl.BlockSpec((1, N1, N1), lambda r, t: (r, 0, 0)),
            pl.BlockSpec((1, 1, N1), lambda r, t: (r, 0, 0)),
            pl.BlockSpec((1, TILE, N1), lambda r, t: (r, t, 0)),
        ],
        out_specs=pl.BlockSpec((TILE, NN), lambda r, t: (r * NT + t, 0)),
        out_shape=jax.ShapeDtypeStruct((NN, NN), jnp.float32),
    )(nv1, nv2, ww, wb, pre)


# speedup vs baseline: 1.0485x; 1.0485x over previous
import jax
import jax.numpy as jnp
from jax.experimental import pallas as pl
from jax.experimental.pallas import tpu as pltpu

N1 = 2048
DIM = 64
K = 20
ALPHA = 3.0
NN = 2 * N1
TILE = 128
NT = N1 // TILE


def _dot_t(a, b, precision=jax.lax.Precision.HIGHEST):
    return jax.lax.dot_general(
        a, b, (((1,), (1,)), ((), ())),
        precision=precision,
        preferred_element_type=jnp.float32)


def _nv_kernel(emb_ref, lw_ref, lb_ref, nv1_ref, nv2_ref):
    for r in range(2):
        for j in range(2):
            i1 = 2 * r + j
            i2 = 2 * j + r
            nv1_ref[r, j] = jnp.tanh(
                ALPHA * (_dot_t(emb_ref[i1], lw_ref[i1]) + lb_ref[i1]))
            nv2_ref[r, j] = jnp.tanh(
                ALPHA * (_dot_t(emb_ref[i2], lw_ref[i1]) + lb_ref[i1]))


def _fused(nv1_ref, nv2_ref, ww_ref, wb_ref, pre_ref, out_ref, adj_ref):
    s = pl.program_id(0)
    nsteps = 2 * NT

    @pl.when(s > 0)
    def _():
        adj = adj_ref[...]
        bits = jax.lax.bitcast_convert_type(adj, jnp.int32)
        lo = jnp.full((TILE, 1), -1, jnp.int32)
        hi = jnp.full((TILE, 1), 0x3F800000, jnp.int32)
        n_gt = jnp.zeros((TILE, 1), jnp.int32)
        for _ in range(31):
            mid = (lo + hi) >> 1
            cnt = jnp.sum((bits > mid).astype(jnp.int32), axis=1,
                          keepdims=True)
            ge = cnt >= K
            lo = jnp.where(ge, mid, lo)
            hi = jnp.where(ge, hi, mid)
            n_gt = jnp.where(ge, n_gt, cnt)
        thr = hi
        m_tie = K - n_gt
        tie = bits == thr
        iota = jax.lax.broadcasted_iota(jnp.int32, (TILE, NN), 1)
        ilo = jnp.full((TILE, 1), -1, jnp.int32)
        ihi = jnp.full((TILE, 1), NN - 1, jnp.int32)
        for _ in range(12):
            mid = (ilo + ihi) >> 1
            c = jnp.sum((tie & (iota <= mid)).astype(jnp.int32), axis=1,
                        keepdims=True)
            ok = c >= m_tie
            ihi = jnp.where(ok, mid, ihi)
            ilo = jnp.where(ok, ilo, mid)
        mask = (bits > thr) | (tie & (iota <= ihi))
        out_ref[...] = jnp.where(mask, adj, 0.0)

    @pl.when(s < nsteps)
    def _():
        t = s % NT
        accw = _dot_t(pre_ref[0], ww_ref[0]) + wb_ref[0]
        row0 = nv1_ref[0, 0, pl.ds(t * TILE, TILE), :]
        row1 = nv1_ref[0, 1, pl.ds(t * TILE, TILE), :]
        log0 = _dot_t(row0, nv2_ref[0, 0]) + accw
        log1 = _dot_t(row1, nv2_ref[0, 1]) + accw
        logits = jnp.concatenate([log0, log1], axis=1)
        adj_ref[...] = jnp.maximum(jnp.tanh(ALPHA * logits), 0.0)


def kernel(emb0, emb1, emb2, emb3, lw0, lw1, lw2, lw3, lb0, lb1, lb2, lb3,
           ww0, ww1, wb0, wb1, pre_adj0, pre_adj1, idx):
    emb = jnp.stack([emb0, emb1, emb2, emb3])
    lw = jnp.stack([lw0, lw1, lw2, lw3])
    lb = jnp.stack([lb0, lb1, lb2, lb3])[:, None, :]
    ww = jnp.stack([ww0, ww1])
    wb = jnp.stack([wb0, wb1])[:, None, :]
    pre = jnp.stack([pre_adj0, pre_adj1])

    nv_shape = jax.ShapeDtypeStruct((2, 2, N1, DIM), jnp.float32)
    nv1, nv2 = pl.pallas_call(
        _nv_kernel,
        out_shape=(nv_shape, nv_shape),
    )(emb, lw, lb)

    def _r(s):
        return jnp.minimum(s // NT, 1)

    return pl.pallas_call(
        _fused,
        grid=(2 * NT + 1,),
        in_specs=[
            pl.BlockSpec((1, 2, N1, DIM), lambda s: (_r(s), 0, 0, 0)),
            pl.BlockSpec((1, 2, N1, DIM), lambda s: (_r(s), 0, 0, 0)),
            pl.BlockSpec((1, N1, N1), lambda s: (_r(s), 0, 0)),
            pl.BlockSpec((1, 1, N1), lambda s: (_r(s), 0, 0)),
            pl.BlockSpec((1, TILE, N1), lambda s: (_r(s), s % NT, 0)),
        ],
        out_specs=pl.BlockSpec(
            (TILE, NN), lambda s: (jnp.maximum(s - 1, 0), 0)),
        out_shape=jax.ShapeDtypeStruct((NN, NN), jnp.float32),
        scratch_shapes=[
            pltpu.VMEM((TILE, NN), jnp.float32),
        ],
    )(nv1, nv2, ww, wb, pre)


# speedup vs baseline: 1.1057x; 1.0546x over previous
import jax
import jax.numpy as jnp
from jax.experimental import pallas as pl
from jax.experimental.pallas import tpu as pltpu

N1 = 2048
DIM = 64
K = 20
ALPHA = 3.0
NN = 2 * N1
TILE = 128
NT = N1 // TILE


def _dot_t(a, b, precision=jax.lax.Precision.HIGHEST):
    return jax.lax.dot_general(
        a, b, (((1,), (1,)), ((), ())),
        precision=precision,
        preferred_element_type=jnp.float32)


def _nv_kernel(emb_ref, lw_ref, lb_ref, nv1_ref, nv2_ref):
    for r in range(2):
        for j in range(2):
            i1 = 2 * r + j
            i2 = 2 * j + r
            nv1_ref[r, j] = jnp.tanh(
                ALPHA * (_dot_t(emb_ref[i1], lw_ref[i1]) + lb_ref[i1]))
            nv2_ref[r, j] = jnp.tanh(
                ALPHA * (_dot_t(emb_ref[i2], lw_ref[i1]) + lb_ref[i1]))


def _fused(nv1_ref, nv2_ref, ww_ref, wb_ref, pre_ref, out_ref, adj_ref):
    s = pl.program_id(0)

    adj = adj_ref[pl.ds(((s + 1) % 2) * TILE, TILE), :]
    bits = jax.lax.bitcast_convert_type(adj, jnp.int32)
    lo = jnp.full((TILE, 1), -1, jnp.int32)
    hi = jnp.full((TILE, 1), 0x3F800000, jnp.int32)
    n_gt = jnp.zeros((TILE, 1), jnp.int32)
    for _ in range(31):
        mid = (lo + hi) >> 1
        cnt = jnp.sum((bits > mid).astype(jnp.int32), axis=1, keepdims=True)
        ge = cnt >= K
        lo = jnp.where(ge, mid, lo)
        hi = jnp.where(ge, hi, mid)
        n_gt = jnp.where(ge, n_gt, cnt)
    thr = hi
    m_tie = K - n_gt
    tie = bits == thr
    iota = jax.lax.broadcasted_iota(jnp.int32, (TILE, NN), 1)
    ilo = jnp.full((TILE, 1), -1, jnp.int32)
    ihi = jnp.full((TILE, 1), NN - 1, jnp.int32)
    for _ in range(12):
        mid = (ilo + ihi) >> 1
        c = jnp.sum((tie & (iota <= mid)).astype(jnp.int32), axis=1,
                    keepdims=True)
        ok = c >= m_tie
        ihi = jnp.where(ok, mid, ihi)
        ilo = jnp.where(ok, ilo, mid)
    mask = (bits > thr) | (tie & (iota <= ihi))
    out_ref[...] = jnp.where(mask, adj, 0.0)

    t = s % NT
    accw = _dot_t(pre_ref[0], ww_ref[0]) + wb_ref[0]
    row0 = nv1_ref[0, 0, pl.ds(t * TILE, TILE), :]
    row1 = nv1_ref[0, 1, pl.ds(t * TILE, TILE), :]
    log0 = _dot_t(row0, nv2_ref[0, 0]) + accw
    log1 = _dot_t(row1, nv2_ref[0, 1]) + accw
    logits = jnp.concatenate([log0, log1], axis=1)
    adj_ref[pl.ds((s % 2) * TILE, TILE), :] = jnp.maximum(
        jnp.tanh(ALPHA * logits), 0.0)


def kernel(emb0, emb1, emb2, emb3, lw0, lw1, lw2, lw3, lb0, lb1, lb2, lb3,
           ww0, ww1, wb0, wb1, pre_adj0, pre_adj1, idx):
    emb = jnp.stack([emb0, emb1, emb2, emb3])
    lw = jnp.stack([lw0, lw1, lw2, lw3])
    lb = jnp.stack([lb0, lb1, lb2, lb3])[:, None, :]
    ww = jnp.stack([ww0, ww1])
    wb = jnp.stack([wb0, wb1])[:, None, :]
    pre = jnp.stack([pre_adj0, pre_adj1])

    nv_shape = jax.ShapeDtypeStruct((2, 2, N1, DIM), jnp.float32)
    nv1, nv2 = pl.pallas_call(
        _nv_kernel,
        out_shape=(nv_shape, nv_shape),
    )(emb, lw, lb)

    def _r(s):
        return jnp.minimum(s // NT, 1)

    return pl.pallas_call(
        _fused,
        grid=(2 * NT + 1,),
        in_specs=[
            pl.BlockSpec((1, 2, N1, DIM), lambda s: (_r(s), 0, 0, 0)),
            pl.BlockSpec((1, 2, N1, DIM), lambda s: (_r(s), 0, 0, 0)),
            pl.BlockSpec((1, N1, N1), lambda s: (_r(s), 0, 0)),
            pl.BlockSpec((1, 1, N1), lambda s: (_r(s), 0, 0)),
            pl.BlockSpec((1, TILE, N1), lambda s: (_r(s), s % NT, 0)),
        ],
        out_specs=pl.BlockSpec(
            (TILE, NN), lambda s: (jnp.maximum(s - 1, 0), 0)),
        out_shape=jax.ShapeDtypeStruct((NN, NN), jnp.float32),
        scratch_shapes=[
            pltpu.VMEM((2 * TILE, NN), jnp.float32),
        ],
    )(nv1, nv2, ww, wb, pre)


# speedup vs baseline: 1.2376x; 1.1193x over previous
import jax
import jax.numpy as jnp
from jax.experimental import pallas as pl
from jax.experimental.pallas import tpu as pltpu

N1 = 2048
DIM = 64
K = 20
ALPHA = 3.0
NN = 2 * N1
TILE = 128
NT = N1 // TILE


def _dot_t(a, b, precision=jax.lax.Precision.HIGHEST):
    return jax.lax.dot_general(
        a, b, (((1,), (1,)), ((), ())),
        precision=precision,
        preferred_element_type=jnp.float32)


def _nv_kernel(emb_ref, lw_ref, lb_ref, nv1_ref, nv2_ref):
    for r in range(2):
        for j in range(2):
            i1 = 2 * r + j
            i2 = 2 * j + r
            nv1_ref[r, j] = jnp.tanh(
                ALPHA * (_dot_t(emb_ref[i1], lw_ref[i1]) + lb_ref[i1]))
            nv2_ref[r, j] = jnp.tanh(
                ALPHA * (_dot_t(emb_ref[i2], lw_ref[i1]) + lb_ref[i1]))


def _fused(nv1_ref, nv2_ref, ww_ref, wb_ref, pre_ref, out_ref, adj_ref):
    s = pl.program_id(0)

    adj = adj_ref[pl.ds(((s + 1) % 2) * TILE, TILE), :]
    bits = jax.lax.bitcast_convert_type(adj, jnp.int32)
    iota = jax.lax.broadcasted_iota(jnp.int32, (TILE, NN), 1)
    B0 = 0x3F7FFF00
    ckey = jnp.where(bits >= B0,
                     (1 << 30) + ((bits - B0) << 12) + (NN - 1) - iota,
                     bits)
    lo = jnp.full((TILE, 1), -1, jnp.int32)
    hi = jnp.full((TILE, 1), (1 << 30) + (1 << 21), jnp.int32)
    for _ in range(31):
        mid = lo + ((hi - lo) >> 1)
        cnt = jnp.sum((ckey > mid).astype(jnp.int32), axis=1, keepdims=True)
        ge = cnt >= K
        lo = jnp.where(ge, mid, lo)
        hi = jnp.where(ge, hi, mid)
    out_ref[...] = jnp.where(ckey >= hi, adj, 0.0)

    t = s % NT
    accw = _dot_t(pre_ref[0], ww_ref[0]) + wb_ref[0]
    row0 = nv1_ref[0, 0, pl.ds(t * TILE, TILE), :]
    row1 = nv1_ref[0, 1, pl.ds(t * TILE, TILE), :]
    log0 = _dot_t(row0, nv2_ref[0, 0]) + accw
    log1 = _dot_t(row1, nv2_ref[0, 1]) + accw
    logits = jnp.concatenate([log0, log1], axis=1)
    adj_ref[pl.ds((s % 2) * TILE, TILE), :] = jnp.maximum(
        jnp.tanh(ALPHA * logits), 0.0)


def kernel(emb0, emb1, emb2, emb3, lw0, lw1, lw2, lw3, lb0, lb1, lb2, lb3,
           ww0, ww1, wb0, wb1, pre_adj0, pre_adj1, idx):
    emb = jnp.stack([emb0, emb1, emb2, emb3])
    lw = jnp.stack([lw0, lw1, lw2, lw3])
    lb = jnp.stack([lb0, lb1, lb2, lb3])[:, None, :]
    ww = jnp.stack([ww0, ww1])
    wb = jnp.stack([wb0, wb1])[:, None, :]
    pre = jnp.stack([pre_adj0, pre_adj1])

    nv_shape = jax.ShapeDtypeStruct((2, 2, N1, DIM), jnp.float32)
    nv1, nv2 = pl.pallas_call(
        _nv_kernel,
        out_shape=(nv_shape, nv_shape),
    )(emb, lw, lb)

    def _r(s):
        return jnp.minimum(s // NT, 1)

    return pl.pallas_call(
        _fused,
        grid=(2 * NT + 1,),
        in_specs=[
            pl.BlockSpec((1, 2, N1, DIM), lambda s: (_r(s), 0, 0, 0)),
            pl.BlockSpec((1, 2, N1, DIM), lambda s: (_r(s), 0, 0, 0)),
            pl.BlockSpec((1, N1, N1), lambda s: (_r(s), 0, 0)),
            pl.BlockSpec((1, 1, N1), lambda s: (_r(s), 0, 0)),
            pl.BlockSpec((1, TILE, N1), lambda s: (_r(s), s % NT, 0)),
        ],
        out_specs=pl.BlockSpec(
            (TILE, NN), lambda s: (jnp.maximum(s - 1, 0), 0)),
        out_shape=jax.ShapeDtypeStruct((NN, NN), jnp.float32),
        scratch_shapes=[
            pltpu.VMEM((2 * TILE, NN), jnp.float32),
        ],
    )(nv1, nv2, ww, wb, pre)


# speedup vs baseline: 1.2899x; 1.0422x over previous
import jax
import jax.numpy as jnp
from jax.experimental import pallas as pl
from jax.experimental.pallas import tpu as pltpu

N1 = 2048
DIM = 64
K = 20
ALPHA = 3.0
NN = 2 * N1
TILE = 128
NT = N1 // TILE


def _dot_t(a, b, precision=jax.lax.Precision.HIGHEST):
    return jax.lax.dot_general(
        a, b, (((1,), (1,)), ((), ())),
        precision=precision,
        preferred_element_type=jnp.float32)


def _nv_kernel(emb_ref, lw_ref, lb_ref, nv1_ref, nv2_ref):
    for r in range(2):
        for j in range(2):
            i1 = 2 * r + j
            i2 = 2 * j + r
            nv1_ref[r, j] = jnp.tanh(
                ALPHA * (_dot_t(emb_ref[i1], lw_ref[i1]) + lb_ref[i1]))
            nv2_ref[r, j] = jnp.tanh(
                ALPHA * (_dot_t(emb_ref[i2], lw_ref[i1]) + lb_ref[i1]))


def _fused(nv1_ref, nv2_ref, ww_ref, wb_ref, pre_ref, out_ref):
    t = pl.program_id(1)

    accw = _dot_t(pre_ref[0], ww_ref[0]) + wb_ref[0]
    row0 = nv1_ref[0, 0, pl.ds(t * TILE, TILE), :]
    row1 = nv1_ref[0, 1, pl.ds(t * TILE, TILE), :]
    log0 = _dot_t(row0, nv2_ref[0, 0]) + accw
    log1 = _dot_t(row1, nv2_ref[0, 1]) + accw
    logits = jnp.concatenate([log0, log1], axis=1)
    adj = jnp.maximum(jnp.tanh(ALPHA * logits), 0.0)

    bits = jax.lax.bitcast_convert_type(adj, jnp.int32)
    iota = jax.lax.broadcasted_iota(jnp.int32, (TILE, NN), 1)
    B0 = 0x3F7FFF00
    ckey = jnp.where(bits >= B0,
                     (1 << 30) + ((bits - B0) << 12) + (NN - 1) - iota,
                     bits)
    lo = jnp.full((TILE, 1), -1, jnp.int32)
    hi = jnp.full((TILE, 1), (1 << 30) + (1 << 21), jnp.int32)
    for _ in range(31):
        mid = lo + ((hi - lo) >> 1)
        cnt = jnp.sum((ckey > mid).astype(jnp.int32), axis=1, keepdims=True)
        ge = cnt >= K
        lo = jnp.where(ge, mid, lo)
        hi = jnp.where(ge, hi, mid)
    out_ref[...] = jnp.where(ckey >= hi, adj, 0.0)


def kernel(emb0, emb1, emb2, emb3, lw0, lw1, lw2, lw3, lb0, lb1, lb2, lb3,
           ww0, ww1, wb0, wb1, pre_adj0, pre_adj1, idx):
    emb = jnp.stack([emb0, emb1, emb2, emb3])
    lw = jnp.stack([lw0, lw1, lw2, lw3])
    lb = jnp.stack([lb0, lb1, lb2, lb3])[:, None, :]
    ww = jnp.stack([ww0, ww1])
    wb = jnp.stack([wb0, wb1])[:, None, :]
    pre = jnp.stack([pre_adj0, pre_adj1])

    nv_shape = jax.ShapeDtypeStruct((2, 2, N1, DIM), jnp.float32)
    nv1, nv2 = pl.pallas_call(
        _nv_kernel,
        out_shape=(nv_shape, nv_shape),
    )(emb, lw, lb)

    return pl.pallas_call(
        _fused,
        grid=(2, NT),
        in_specs=[
            pl.BlockSpec((1, 2, N1, DIM), lambda r, t: (r, 0, 0, 0)),
            pl.BlockSpec((1, 2, N1, DIM), lambda r, t: (r, 0, 0, 0)),
            pl.BlockSpec((1, N1, N1), lambda r, t: (r, 0, 0)),
            pl.BlockSpec((1, 1, N1), lambda r, t: (r, 0, 0)),
            pl.BlockSpec((1, TILE, N1), lambda r, t: (r, t, 0)),
        ],
        out_specs=pl.BlockSpec((TILE, NN), lambda r, t: (r * NT + t, 0)),
        out_shape=jax.ShapeDtypeStruct((NN, NN), jnp.float32),
    )(nv1, nv2, ww, wb, pre)
